# baseline (speedup 1.0000x reference)
import functools

import jax
import jax.numpy as jnp
from jax import lax
from jax.experimental import pallas as pl
from jax.experimental.pallas import tpu as pltpu
from jax.experimental.pallas import tpu_sc as plsc

HIDDEN_DIM = 768
N_EXPERTS = 8
N_TOKENS = 32768

TILE = 2048

NUM_CORES = 2
NUM_SUBCORES = 16
NUM_WORKERS = NUM_CORES * NUM_SUBCORES
LANES = 16
LANE = 128

N_BLOCKS = N_TOKENS // LANE
TOK_PER_W = N_TOKENS // NUM_WORKERS
BLK_W = N_BLOCKS // NUM_WORKERS
GRP_PER_BLK = LANE // LANES


def _gate_body(hs_ref, w_ref, b_ref, out_ref):
    logits = lax.dot_general(
        w_ref[...], hs_ref[...],
        dimension_numbers=(((1,), (1,)), ((), ())),
        preferred_element_type=jnp.float32,
    ) + b_ref[...]
    for t in range(TILE // LANE):
        out_ref[t] = logits[:, t * LANE:(t + 1) * LANE]


_gate = pl.pallas_call(
    _gate_body,
    grid=(N_TOKENS // TILE,),
    in_specs=[
        pl.BlockSpec((TILE, HIDDEN_DIM), lambda i: (i, 0)),
        pl.BlockSpec((N_EXPERTS, HIDDEN_DIM), lambda i: (0, 0)),
        pl.BlockSpec((N_EXPERTS, 1), lambda i: (0, 0)),
    ],
    out_specs=pl.BlockSpec(
        (TILE // LANE, N_EXPERTS, LANE), lambda i: (i, 0, 0)),
    out_shape=jax.ShapeDtypeStruct(
        (N_BLOCKS, N_EXPERTS, LANE), jnp.float32),
)


_sc_mesh = plsc.VectorSubcoreMesh(
    core_axis_name="c", subcore_axis_name="s",
    num_cores=NUM_CORES, num_subcores=NUM_SUBCORES,
)


@functools.partial(
    pl.kernel,
    out_type=[
        jax.ShapeDtypeStruct((2 * N_TOKENS,), jnp.float32),
        jax.ShapeDtypeStruct((2 * N_TOKENS,), jnp.int32),
    ],
    mesh=_sc_mesh,
    compiler_params=pltpu.CompilerParams(needs_layout_passes=False),
    scratch_types=[
        pltpu.VMEM((BLK_W, N_EXPERTS, LANE), jnp.float32),
        pltpu.VMEM((2 * TOK_PER_W,), jnp.float32),
        pltpu.VMEM((2 * TOK_PER_W,), jnp.int32),
        pltpu.SemaphoreType.DMA,
    ],
)
def _router(logits_hbm, w_hbm, i_hbm, lbuf, wpair, ipair, sem):
    wid = lax.axis_index("s") * NUM_CORES + lax.axis_index("c")
    base = wid * TOK_PER_W

    pltpu.async_copy(
        logits_hbm.at[pl.ds(wid * BLK_W, BLK_W)], lbuf, sem,
    ).wait()

    lane_iota = lax.iota(jnp.int32, LANES)

    for b in range(BLK_W):
        def body(g, _, b=b):
            tb = g * LANES

            m1 = lbuf[b, 0, pl.ds(tb, LANES)]
            i1 = jnp.zeros((LANES,), jnp.int32)
            m2 = jnp.full((LANES,), -jnp.inf, jnp.float32)
            i2 = jnp.zeros((LANES,), jnp.int32)
            for e in range(1, N_EXPERTS):
                v = lbuf[b, e, pl.ds(tb, LANES)]
                evec = jnp.full((LANES,), e, jnp.int32)
                gt1 = v > m1
                gt2 = v > m2
                i2 = jnp.where(gt1, i1, jnp.where(gt2, evec, i2))
                m2 = jnp.where(gt1, m1, jnp.where(gt2, v, m2))
                i1 = jnp.where(gt1, evec, i1)
                m1 = jnp.where(gt1, v, m1)

            ex = jnp.exp(m2 - m1)
            r = 1.0 / (1.0 + ex)
            pos = (b * LANE + tb + lane_iota) * 2
            plsc.store_scatter(wpair, [pos], r)
            plsc.store_scatter(wpair, [pos + 1], ex * r)
            plsc.store_scatter(ipair, [pos], i1)
            plsc.store_scatter(ipair, [pos + 1], i2)
            return 0

        lax.fori_loop(0, GRP_PER_BLK, body, 0)

    pltpu.sync_copy(wpair, w_hbm.at[pl.ds(2 * base, 2 * TOK_PER_W)])
    pltpu.sync_copy(ipair, i_hbm.at[pl.ds(2 * base, 2 * TOK_PER_W)])


def kernel(hidden_states, gate_w, gate_b):
    logits3d = _gate(hidden_states, gate_w, gate_b.reshape(N_EXPERTS, 1))
    wflat, iflat = _router(logits3d)
    weights = wflat.reshape(N_TOKENS, 2)
    topk_idx = iflat.reshape(N_TOKENS, 2)
    return (weights, topk_idx)


# speedup vs baseline: 1.9200x; 1.9200x over previous
import functools

import jax
import jax.numpy as jnp
from jax import lax
from jax.experimental import pallas as pl
from jax.experimental.pallas import tpu as pltpu
from jax.experimental.pallas import tpu_sc as plsc

HIDDEN_DIM = 768
N_EXPERTS = 8
N_TOKENS = 32768

TILE = 2048

NUM_CORES = 2
NUM_SUBCORES = 16
NUM_WORKERS = NUM_CORES * NUM_SUBCORES
LANES = 16
LANE = 128

N_BLOCKS = N_TOKENS // LANE
TOK_PER_W = N_TOKENS // NUM_WORKERS
BLK_W = N_BLOCKS // NUM_WORKERS
GRP_PER_BLK = LANE // LANES


def _gate_body(hs_ref, w_ref, b_ref, out_ref):
    logits = lax.dot_general(
        w_ref[...], hs_ref[...],
        dimension_numbers=(((1,), (1,)), ((), ())),
        preferred_element_type=jnp.float32,
    ) + b_ref[...]
    for t in range(TILE // LANE):
        out_ref[t] = logits[:, t * LANE:(t + 1) * LANE]


_gate = pl.pallas_call(
    _gate_body,
    grid=(N_TOKENS // TILE,),
    in_specs=[
        pl.BlockSpec((TILE, HIDDEN_DIM), lambda i: (i, 0)),
        pl.BlockSpec((N_EXPERTS, HIDDEN_DIM), lambda i: (0, 0)),
        pl.BlockSpec((N_EXPERTS, 1), lambda i: (0, 0)),
    ],
    out_specs=pl.BlockSpec(
        (TILE // LANE, N_EXPERTS, LANE), lambda i: (i, 0, 0)),
    out_shape=jax.ShapeDtypeStruct(
        (N_BLOCKS, N_EXPERTS, LANE), jnp.float32),
)


_sc_mesh = plsc.VectorSubcoreMesh(
    core_axis_name="c", subcore_axis_name="s",
    num_cores=NUM_CORES, num_subcores=NUM_SUBCORES,
)


@functools.partial(
    pl.kernel,
    out_type=[
        jax.ShapeDtypeStruct((N_TOKENS,), jnp.float32),
        jax.ShapeDtypeStruct((N_TOKENS,), jnp.float32),
        jax.ShapeDtypeStruct((N_TOKENS,), jnp.int32),
        jax.ShapeDtypeStruct((N_TOKENS,), jnp.int32),
    ],
    mesh=_sc_mesh,
    compiler_params=pltpu.CompilerParams(needs_layout_passes=False),
    scratch_types=[
        pltpu.VMEM((BLK_W, N_EXPERTS, LANE), jnp.float32),
        pltpu.VMEM((TOK_PER_W,), jnp.float32),
        pltpu.VMEM((TOK_PER_W,), jnp.float32),
        pltpu.VMEM((TOK_PER_W,), jnp.int32),
        pltpu.VMEM((TOK_PER_W,), jnp.int32),
        pltpu.SemaphoreType.DMA,
    ],
)
def _router(logits_hbm, w1_hbm, w2_hbm, i1_hbm, i2_hbm,
            lbuf, w1v, w2v, i1v, i2v, sem):
    wid = lax.axis_index("s") * NUM_CORES + lax.axis_index("c")
    base = wid * TOK_PER_W

    pltpu.async_copy(
        logits_hbm.at[pl.ds(wid * BLK_W, BLK_W)], lbuf, sem,
    ).wait()

    for b in range(BLK_W):
        def body(g, _, b=b):
            tb = g * LANES

            m1 = lbuf[b, 0, pl.ds(tb, LANES)]
            i1 = jnp.zeros((LANES,), jnp.int32)
            m2 = jnp.full((LANES,), -jnp.inf, jnp.float32)
            i2 = jnp.zeros((LANES,), jnp.int32)
            for e in range(1, N_EXPERTS):
                v = lbuf[b, e, pl.ds(tb, LANES)]
                evec = jnp.full((LANES,), e, jnp.int32)
                gt1 = v > m1
                gt2 = v > m2
                i2 = jnp.where(gt1, i1, jnp.where(gt2, evec, i2))
                m2 = jnp.where(gt1, m1, jnp.where(gt2, v, m2))
                i1 = jnp.where(gt1, evec, i1)
                m1 = jnp.where(gt1, v, m1)

            ex = jnp.exp(m2 - m1)
            r = 1.0 / (1.0 + ex)
            w1v[pl.ds(b * LANE + tb, LANES)] = r
            w2v[pl.ds(b * LANE + tb, LANES)] = ex * r
            i1v[pl.ds(b * LANE + tb, LANES)] = i1
            i2v[pl.ds(b * LANE + tb, LANES)] = i2
            return 0

        lax.fori_loop(0, GRP_PER_BLK, body, 0)

    pltpu.sync_copy(w1v, w1_hbm.at[pl.ds(base, TOK_PER_W)])
    pltpu.sync_copy(w2v, w2_hbm.at[pl.ds(base, TOK_PER_W)])
    pltpu.sync_copy(i1v, i1_hbm.at[pl.ds(base, TOK_PER_W)])
    pltpu.sync_copy(i2v, i2_hbm.at[pl.ds(base, TOK_PER_W)])


def kernel(hidden_states, gate_w, gate_b):
    logits3d = _gate(hidden_states, gate_w, gate_b.reshape(N_EXPERTS, 1))
    w1, w2, i1, i2 = _router(logits3d)
    weights = jnp.stack([w1, w2], axis=1)
    topk_idx = jnp.stack([i1, i2], axis=1)
    return (weights, topk_idx)


# speedup vs baseline: 1.9483x; 1.0147x over previous
import functools

import jax
import jax.numpy as jnp
from jax import lax
from jax.experimental import pallas as pl
from jax.experimental.pallas import tpu as pltpu
from jax.experimental.pallas import tpu_sc as plsc

HIDDEN_DIM = 768
N_EXPERTS = 8
N_TOKENS = 32768

TILE = 2048

NUM_CORES = 2
NUM_SUBCORES = 16
NUM_WORKERS = NUM_CORES * NUM_SUBCORES
LANES = 16
LANE = 128

N_BLOCKS = N_TOKENS // LANE
TOK_PER_W = N_TOKENS // NUM_WORKERS
BLK_W = N_BLOCKS // NUM_WORKERS
GRP_PER_BLK = LANE // LANES


def _gate_body(hs_ref, w_ref, b_ref, out_ref):
    logits = lax.dot_general(
        w_ref[...], hs_ref[...],
        dimension_numbers=(((1,), (1,)), ((), ())),
        preferred_element_type=jnp.float32,
    ) + b_ref[...]
    for t in range(TILE // LANE):
        out_ref[t] = logits[:, t * LANE:(t + 1) * LANE]


_gate = pl.pallas_call(
    _gate_body,
    grid=(N_TOKENS // TILE,),
    in_specs=[
        pl.BlockSpec((TILE, HIDDEN_DIM), lambda i: (i, 0)),
        pl.BlockSpec((N_EXPERTS, HIDDEN_DIM), lambda i: (0, 0)),
        pl.BlockSpec((N_EXPERTS, 1), lambda i: (0, 0)),
    ],
    out_specs=pl.BlockSpec(
        (TILE // LANE, N_EXPERTS, LANE), lambda i: (i, 0, 0)),
    out_shape=jax.ShapeDtypeStruct(
        (N_BLOCKS, N_EXPERTS, LANE), jnp.float32),
)


_sc_mesh = plsc.VectorSubcoreMesh(
    core_axis_name="c", subcore_axis_name="s",
    num_cores=NUM_CORES, num_subcores=NUM_SUBCORES,
)


@functools.partial(
    pl.kernel,
    out_type=[
        jax.ShapeDtypeStruct((N_TOKENS,), jnp.float32),
        jax.ShapeDtypeStruct((N_TOKENS,), jnp.float32),
        jax.ShapeDtypeStruct((N_TOKENS,), jnp.int32),
        jax.ShapeDtypeStruct((N_TOKENS,), jnp.int32),
    ],
    mesh=_sc_mesh,
    compiler_params=pltpu.CompilerParams(needs_layout_passes=False),
    scratch_types=[
        pltpu.VMEM((BLK_W, N_EXPERTS, LANE), jnp.float32),
        pltpu.VMEM((TOK_PER_W,), jnp.float32),
        pltpu.VMEM((TOK_PER_W,), jnp.float32),
        pltpu.VMEM((TOK_PER_W,), jnp.int32),
        pltpu.VMEM((TOK_PER_W,), jnp.int32),
        pltpu.SemaphoreType.DMA,
    ],
)
def _router(logits_hbm, w1_hbm, w2_hbm, i1_hbm, i2_hbm,
            lbuf, w1v, w2v, i1v, i2v, sem):
    wid = lax.axis_index("s") * NUM_CORES + lax.axis_index("c")
    base = wid * TOK_PER_W

    pltpu.async_copy(
        logits_hbm.at[pl.ds(wid * BLK_W, BLK_W)], lbuf, sem,
    ).wait()

    def body(t, _):
        b = t // GRP_PER_BLK
        tb = (t % GRP_PER_BLK) * LANES

        m1 = lbuf[b, 0, pl.ds(tb, LANES)]
        i1 = jnp.zeros((LANES,), jnp.int32)
        m2 = jnp.full((LANES,), -jnp.inf, jnp.float32)
        i2 = jnp.zeros((LANES,), jnp.int32)
        for e in range(1, N_EXPERTS):
            v = lbuf[b, e, pl.ds(tb, LANES)]
            evec = jnp.full((LANES,), e, jnp.int32)
            gt1 = v > m1
            gt2 = v > m2
            i2 = jnp.where(gt1, i1, jnp.where(gt2, evec, i2))
            m2 = jnp.where(gt1, m1, jnp.where(gt2, v, m2))
            i1 = jnp.where(gt1, evec, i1)
            m1 = jnp.where(gt1, v, m1)

        ex = jnp.exp(m2 - m1)
        r = 1.0 / (1.0 + ex)
        w1v[pl.ds(t * LANES, LANES)] = r
        w2v[pl.ds(t * LANES, LANES)] = ex * r
        i1v[pl.ds(t * LANES, LANES)] = i1
        i2v[pl.ds(t * LANES, LANES)] = i2
        return 0

    lax.fori_loop(0, BLK_W * GRP_PER_BLK, body, 0)

    pltpu.sync_copy(w1v, w1_hbm.at[pl.ds(base, TOK_PER_W)])
    pltpu.sync_copy(w2v, w2_hbm.at[pl.ds(base, TOK_PER_W)])
    pltpu.sync_copy(i1v, i1_hbm.at[pl.ds(base, TOK_PER_W)])
    pltpu.sync_copy(i2v, i2_hbm.at[pl.ds(base, TOK_PER_W)])


def kernel(hidden_states, gate_w, gate_b):
    logits3d = _gate(hidden_states, gate_w, gate_b.reshape(N_EXPERTS, 1))
    w1, w2, i1, i2 = _router(logits3d)
    weights = jnp.stack([w1, w2], axis=1)
    topk_idx = jnp.stack([i1, i2], axis=1)
    return (weights, topk_idx)
